# baseline (speedup 1.0000x reference)
import functools

import jax
import jax.numpy as jnp
from jax import lax
from jax.experimental import pallas as pl
from jax.experimental.pallas import tpu as pltpu
from jax.experimental.pallas import tpu_sc as plsc

CAPACITY = 1_000_000
ROW = 67
BATCH = 16384

G = 16
GPS = 6
NG = CAPACITY * ROW // G
_NW = 32
_BPW = BATCH // _NW
_NCHUNK = 4
_CS = _BPW // _NCHUNK

_mesh = plsc.VectorSubcoreMesh(core_axis_name="c", subcore_axis_name="s")


@functools.partial(
    pl.kernel,
    mesh=_mesh,
    out_type=jax.ShapeDtypeStruct((BATCH, ROW), jnp.float32),
    scratch_types=[
        pltpu.VMEM((_BPW,), jnp.int32),
        pltpu.VMEM((_BPW,), jnp.int32),
        pltpu.VMEM((_BPW * GPS,), jnp.int32),

        [pltpu.VMEM((_CS * GPS, G), jnp.float32) for _ in range(2)],
        pltpu.VMEM((_BPW, ROW), jnp.float32),
        [pltpu.SemaphoreType.DMA for _ in range(2)],
    ],
    compiler_params=pltpu.CompilerParams(
        use_tc_tiling_on_sc=False, needs_layout_passes=False),
)
def _sample(mem16, idx_hbm, out_hbm, idx_v, r_v, gidx, raw, out_v, sem):
    wid = lax.axis_index("s") * 2 + lax.axis_index("c")
    base = wid * _BPW
    pltpu.sync_copy(idx_hbm.at[pl.ds(base, _BPW)], idx_v)

    lanevec = lax.iota(jnp.int32, 16)

    def gen(m, _):
        i16 = idx_v[pl.ds(m * 16, 16)]
        e = i16 * ROW
        s = lax.shift_right_logical(e, 4)
        r_v[pl.ds(m * 16, 16)] = lax.bitwise_and(e, 15)
        tgt = (m * 16 + lanevec) * GPS
        for j in range(GPS):
            plsc.store_scatter(gidx, [tgt + j], jnp.minimum(s + j, NG - 1))
        return _

    lax.fori_loop(0, _BPW // 16, gen, 0, unroll=2)

    def fire(ch):
        return pltpu.async_copy(
            mem16.at[gidx.at[pl.ds(ch * _CS * GPS, _CS * GPS)]],
            raw[ch % 2], sem[ch % 2])

    handles = {0: fire(0)}

    def realign_chunk(ch):
        buf = raw[ch % 2]

        def body(m, _):
            kk0 = ch * _CS + m * 16
            r16 = r_v[pl.ds(kk0, 16)]
            for l in range(16):
                kk = kk0 + l
                fb = (m * 16 + l) * (GPS * G) + r16[l]
                for c in (0, 16, 32, 48, 51):
                    f = fb + c + lanevec
                    row = lax.shift_right_logical(f, 4)
                    col = lax.bitwise_and(f, 15)
                    out_v[kk, pl.ds(c, 16)] = plsc.load_gather(buf, [row, col])
            return _

        lax.fori_loop(0, _CS // 16, body, 0)

    for ch in range(_NCHUNK):
        if ch + 1 < _NCHUNK:
            handles[ch + 1] = fire(ch + 1)
        handles[ch].wait()
        realign_chunk(ch)

    pltpu.sync_copy(out_v, out_hbm.at[pl.ds(base, _BPW)])


def kernel(memory, indices):
    mem16 = memory.reshape(NG, G)
    return _sample(mem16, indices)


# speedup vs baseline: 2.6827x; 2.6827x over previous
import functools

import jax
import jax.numpy as jnp
from jax import lax
from jax.experimental import pallas as pl
from jax.experimental.pallas import tpu as pltpu
from jax.experimental.pallas import tpu_sc as plsc

CAPACITY = 1_000_000
ROW = 67
BATCH = 16384

_NW = 32
_BPW = BATCH // _NW

_mesh = plsc.VectorSubcoreMesh(core_axis_name="c", subcore_axis_name="s")


@functools.partial(
    pl.kernel,
    mesh=_mesh,
    out_type=jax.ShapeDtypeStruct((BATCH, ROW), jnp.float32),
    scratch_types=[
        pltpu.VMEM((_BPW,), jnp.int32),
        pltpu.SemaphoreType.DMA,
    ],
    compiler_params=pltpu.CompilerParams(needs_layout_passes=False),
)
def _sample(mem_hbm, idx_hbm, out_hbm, idx_v, sem):
    wid = lax.axis_index("s") * 2 + lax.axis_index("c")
    base = wid * _BPW
    pltpu.sync_copy(idx_hbm.at[pl.ds(base, _BPW)], idx_v)

    def fire(m, _):
        i16 = idx_v[pl.ds(m * 16, 16)]
        for l in range(16):
            k = m * 16 + l
            pltpu.async_copy(mem_hbm.at[i16[l]], out_hbm.at[base + k], sem)
        return _

    lax.fori_loop(0, _BPW // 16, fire, 0)

    def drain(m, _):
        pltpu.make_async_copy(mem_hbm.at[0], out_hbm.at[base], sem).wait()
        return _

    lax.fori_loop(0, _BPW, drain, 0)


def kernel(memory, indices):
    return _sample(memory, indices)
